# baseline (speedup 1.0000x reference)
import functools

import jax
import jax.numpy as jnp
from jax import lax
from jax.experimental import pallas as pl
from jax.experimental.pallas import tpu as pltpu
from jax.experimental.pallas import tpu_sc as plsc

N = 10000
D = 512
DW = D // 2
B = 3500
K1 = 32
K2 = 64
NB = 7
BOT = 256
BP = 3584
HBA = BP
HBB = BP - HBA
NW = 32
CHN = 56
NBUF = 4
BB1 = 56
BB2 = 448
EPS = 1e-8
PER_CLASS = B // NB


@functools.cache
def _sc_gather_build(hb, cen_rows):
    nch = (hb * (K1 + K2) // NW) // CHN
    cch = cen_rows // (NW * CHN)
    assert nch % NBUF == 0
    mesh = plsc.VectorSubcoreMesh(core_axis_name="c", subcore_axis_name="s")
    out_type = [jax.ShapeDtypeStruct((hb * (K1 + K2), DW), jnp.int32)]
    scratch = [
        pltpu.VMEM((nch, CHN), jnp.int32),
        pltpu.VMEM((NBUF, CHN, DW), jnp.int32),
        [pltpu.SemaphoreType.DMA] * NBUF,
        [pltpu.SemaphoreType.DMA] * NBUF,
    ]
    if cch:
        out_type.append(jax.ShapeDtypeStruct((cen_rows, D), jnp.float32))
        scratch += [
            pltpu.VMEM((cch, CHN), jnp.int32),
            pltpu.VMEM((CHN, D), jnp.float32),
            pltpu.SemaphoreType.DMA,
        ]

    @functools.partial(pl.kernel, mesh=mesh, out_type=out_type,
                       scratch_types=scratch)
    def _sc_gather(table_i32, table_f32, idxn2d, *rest):
        if cch:
            (idxc2d, out_nbr, out_cen,
             idx_v, rows_v, sem_g, sem_o, cidx_v, cen_v, sem_c) = rest
        else:
            (out_nbr, idx_v, rows_v, sem_g, sem_o) = rest
        w = lax.axis_index("s") * 2 + lax.axis_index("c")
        base = w * nch
        pltpu.sync_copy(idxn2d.at[pl.ds(base, nch)], idx_v)
        if cch:
            pltpu.sync_copy(idxc2d.at[pl.ds(w * cch, cch)], cidx_v)

        def issue_gather(l, b):
            pltpu.async_copy(table_i32.at[idx_v.at[l]], rows_v.at[b], sem_g[b])

        def wait_gather(b):
            pltpu.make_async_copy(table_i32.at[idx_v.at[0]], rows_v.at[b],
                                  sem_g[b]).wait()

        def put(l, b):
            pltpu.async_copy(rows_v.at[b],
                             out_nbr.at[pl.ds((base + l) * CHN, CHN)],
                             sem_o[b])

        def drain_out(b):
            pltpu.make_async_copy(rows_v.at[b], out_nbr.at[pl.ds(0, CHN)],
                                  sem_o[b]).wait()

        for b in range(NBUF):
            issue_gather(b, b)

        def step(i, carry):
            first = NBUF * i
            for b in range(NBUF):
                l = first + b
                wait_gather(b)
                put(l, b)
                nxt = l + NBUF

                @pl.when(nxt < nch)
                def _next():
                    drain_out(b)
                    issue_gather(nxt, b)

            return carry

        lax.fori_loop(0, nch // NBUF, step, 0)
        for b in range(NBUF):
            drain_out(b)

        for t in range(cch):
            pltpu.async_copy(table_f32.at[cidx_v.at[t]], cen_v, sem_c).wait()
            pltpu.sync_copy(cen_v,
                            out_cen.at[pl.ds((w * cch + t) * CHN, CHN)])

    return _sc_gather


def _agg_body(cen_ref, g1_ref, g2_ref, lab_ref, ws_ref, wn_ref, wn2_ref,
              w1_ref, b1_ref, w2_ref, b2_ref, raw_ref, csum_ref):
    i = pl.program_id(0)
    c = ws_ref[...] * cen_ref[...]

    def unpack(x):
        lo = lax.bitcast_convert_type(x << 16, jnp.float32)
        hi = lax.bitcast_convert_type(x & jnp.int32(-65536), jnp.float32)
        return jnp.concatenate([lo, hi], axis=-1)

    g1 = wn_ref[...][:, None, :] * unpack(g1_ref[...]).reshape(BB1, K1, D)
    g2 = wn2_ref[...][:, None, :] * unpack(g2_ref[...]).reshape(BB1, K2, D)
    na = jnp.maximum(jnp.sqrt(jnp.sum(c * c, axis=-1)), EPS)
    n1 = jnp.maximum(jnp.sqrt(jnp.sum(g1 * g1, axis=-1)), EPS)
    n2 = jnp.maximum(jnp.sqrt(jnp.sum(g2 * g2, axis=-1)), EPS)
    d1 = jnp.sum(c[:, None, :] * g1, axis=-1)
    d2 = jnp.sum(c[:, None, :] * g2, axis=-1)
    s1 = d1 / (na[:, None] * n1)
    s2 = d2 / (na[:, None] * n2)
    m = jnp.maximum(jnp.max(s1, axis=-1), jnp.max(s2, axis=-1))
    e1 = jnp.exp(s1 - m[:, None])
    e2 = jnp.exp(s2 - m[:, None])
    z = jnp.sum(e1, axis=-1) + jnp.sum(e2, axis=-1)
    p1 = e1 / z[:, None]
    p2 = e2 / z[:, None]
    wsum = (jnp.sum(p1[:, :, None] * g1, axis=1)
            + jnp.sum(p2[:, :, None] * g2, axis=1))
    x = wsum + c
    h = jnp.maximum(
        jnp.dot(x, w1_ref[...], preferred_element_type=jnp.float32)
        + b1_ref[...], 0.0)
    pr = jnp.dot(h, w2_ref[...], preferred_element_type=jnp.float32) + b2_ref[...]
    raw = pr + c
    raw_ref[...] = raw
    lab = lab_ref[0]
    cls = lax.broadcasted_iota(jnp.int32, (8, BB1), 0)
    pmat = (cls == lab).astype(jnp.float32)
    part = jnp.dot(pmat, raw, preferred_element_type=jnp.float32)

    @pl.when(i == 0)
    def _init():
        csum_ref[...] = jnp.zeros_like(csum_ref)

    csum_ref[...] += part


def _agg_call(gnbr, cen, cen_blk, hb, lab3, w_self, w_nbr, w_nbr2,
              W1, b1, W2, b2):
    full2 = lambda shape: pl.BlockSpec(shape, lambda i: (0, 0))
    n1_blk = (hb * K2) // (BB1 * K1)
    return pl.pallas_call(
        _agg_body,
        grid=(hb // BB1,),
        in_specs=[
            pl.BlockSpec((BB1, D), lambda i: (cen_blk + i, 0)),
            pl.BlockSpec((BB1 * K1, DW), lambda i: (n1_blk + i, 0)),
            pl.BlockSpec((BB1 * K2, DW), lambda i: (i, 0)),
            pl.BlockSpec((1, 1, BB1), lambda i: (i, 0, 0)),
            full2((1, D)), full2((1, D)), full2((1, D)),
            full2((D, BOT)), full2((1, BOT)), full2((BOT, D)), full2((1, D)),
        ],
        out_specs=[
            pl.BlockSpec((BB1, D), lambda i: (i, 0)),
            pl.BlockSpec((8, D), lambda i: (0, 0)),
        ],
        out_shape=[
            jax.ShapeDtypeStruct((hb, D), jnp.float32),
            jax.ShapeDtypeStruct((8, D), jnp.float32),
        ],
    )(cen, gnbr, gnbr, lab3, w_self, w_nbr, w_nbr2, W1, b1, W2, b2)


def _cos_body(raw_ref, csa_ref, csb_ref, o_ref):
    ave = (csa_ref[...] + csb_ref[...]) * (1.0 / PER_CLASS)
    r = raw_ref[...]
    dots = lax.dot_general(r, ave, (((1,), (1,)), ((), ())),
                           preferred_element_type=jnp.float32)
    na = jnp.maximum(jnp.sqrt(jnp.sum(r * r, axis=-1)), EPS)
    nb = jnp.maximum(jnp.sqrt(jnp.sum(ave * ave, axis=-1)), EPS)
    sim = dots / (na[:, None] * nb[None, :])
    col = lax.broadcasted_iota(jnp.int32, (BB2, 8), 1)
    sim = jnp.where(col < NB, sim, -1e30)
    m = jnp.max(sim, axis=-1)
    e = jnp.exp(sim - m[:, None])
    o_ref[...] = e / jnp.sum(e, axis=-1)[:, None]


def _cos_call(raw, csum_a, csum_b, hb):
    return pl.pallas_call(
        _cos_body,
        grid=(hb // BB2,),
        in_specs=[
            pl.BlockSpec((BB2, D), lambda i: (i, 0)),
            pl.BlockSpec((8, D), lambda i: (0, 0)),
            pl.BlockSpec((8, D), lambda i: (0, 0)),
        ],
        out_specs=pl.BlockSpec((BB2, 8), lambda i: (i, 0)),
        out_shape=jax.ShapeDtypeStruct((hb, 8), jnp.float32),
    )(raw, csum_a, csum_b)


def kernel(embeds, idx, neighbors, neighbors_2hop, labels, w_self, w_nbr,
           w_nbr2, W1, b1, W2, b2):
    pad = BP - B
    idxp = jnp.concatenate([idx, jnp.zeros((pad,), jnp.int32)])
    nbrp = jnp.concatenate([neighbors, jnp.zeros((pad, K1), jnp.int32)])
    nbr2p = jnp.concatenate([neighbors_2hop, jnp.zeros((pad, K2), jnp.int32)])
    labp = jnp.concatenate([labels, jnp.full((pad,), NB, jnp.int32)])

    emb_bf = embeds.astype(jnp.bfloat16)
    emb_i32 = lax.bitcast_convert_type(
        jnp.stack([emb_bf[:, :DW], emb_bf[:, DW:]], axis=-1), jnp.int32)

    b1r = b1.reshape(1, BOT)
    b2r = b2.reshape(1, D)

    def nbr_idx(lo, hb):
        return jnp.concatenate(
            [nbr2p[lo:lo + hb].reshape(hb * K2),
             nbrp[lo:lo + hb].reshape(hb * K1)]).reshape(-1, CHN)

    idxc2d = idxp.reshape(BP // CHN, CHN)
    gnbr_a, cen = _sc_gather_build(HBA, BP)(
        emb_i32, embeds, nbr_idx(0, HBA), idxc2d)
    lab3_a = labp[:HBA].reshape(HBA // BB1, 1, BB1)
    raw_a, cs_a = _agg_call(gnbr_a, cen, 0, HBA, lab3_a, w_self, w_nbr,
                            w_nbr2, W1, b1r, W2, b2r)
    if HBB:
        res_b = _sc_gather_build(HBB, 0)(emb_i32, embeds, nbr_idx(HBA, HBB))
        gnbr_b = res_b[0] if isinstance(res_b, (list, tuple)) else res_b
        lab3_b = labp[HBA:].reshape(HBB // BB1, 1, BB1)
        raw_b, cs_b = _agg_call(gnbr_b, cen, HBA // BB1, HBB, lab3_b, w_self,
                                w_nbr, w_nbr2, W1, b1r, W2, b2r)
        out_a = _cos_call(raw_a, cs_a, cs_b, HBA)
        out_b = _cos_call(raw_b, cs_a, cs_b, HBB)
        out = jnp.concatenate([out_a, out_b], axis=0)
    else:
        out = _cos_call(raw_a, cs_a, jnp.zeros((8, D), jnp.float32), HBA)
    return out[:B, :NB]


# speedup vs baseline: 1.3200x; 1.3200x over previous
import functools

import jax
import jax.numpy as jnp
from jax import lax
from jax.experimental import pallas as pl
from jax.experimental.pallas import tpu as pltpu
from jax.experimental.pallas import tpu_sc as plsc

N = 10000
D = 512
DW = D // 2
B = 3500
K1 = 32
K2 = 64
NB = 7
BOT = 256
BP = 3584
HBA = 2688
HBB = BP - HBA
NW = 32
CHN = 56
NBUF = 4
BB1 = 56
BB2 = 448
EPS = 1e-8
PER_CLASS = B // NB


@functools.cache
def _sc_gather_build(hb, cen_rows):
    nch = (hb * (K1 + K2) // NW) // CHN
    cch = cen_rows // (NW * CHN)
    assert nch % NBUF == 0
    mesh = plsc.VectorSubcoreMesh(core_axis_name="c", subcore_axis_name="s")
    out_type = [jax.ShapeDtypeStruct((hb * (K1 + K2), DW), jnp.int32)]
    scratch = [
        pltpu.VMEM((nch, CHN), jnp.int32),
        pltpu.VMEM((NBUF, CHN, DW), jnp.int32),
        [pltpu.SemaphoreType.DMA] * NBUF,
        [pltpu.SemaphoreType.DMA] * NBUF,
    ]
    if cch:
        out_type.append(jax.ShapeDtypeStruct((cen_rows, D), jnp.float32))
        scratch += [
            pltpu.VMEM((cch, CHN), jnp.int32),
            pltpu.VMEM((CHN, D), jnp.float32),
            pltpu.SemaphoreType.DMA,
        ]

    @functools.partial(pl.kernel, mesh=mesh, out_type=out_type,
                       scratch_types=scratch)
    def _sc_gather(table_i32, table_f32, idxn2d, *rest):
        if cch:
            (idxc2d, out_nbr, out_cen,
             idx_v, rows_v, sem_g, sem_o, cidx_v, cen_v, sem_c) = rest
        else:
            (out_nbr, idx_v, rows_v, sem_g, sem_o) = rest
        w = lax.axis_index("s") * 2 + lax.axis_index("c")
        base = w * nch
        pltpu.sync_copy(idxn2d.at[pl.ds(base, nch)], idx_v)
        if cch:
            pltpu.sync_copy(idxc2d.at[pl.ds(w * cch, cch)], cidx_v)

        def issue_gather(l, b):
            pltpu.async_copy(table_i32.at[idx_v.at[l]], rows_v.at[b], sem_g[b])

        def wait_gather(b):
            pltpu.make_async_copy(table_i32.at[idx_v.at[0]], rows_v.at[b],
                                  sem_g[b]).wait()

        def put(l, b):
            pltpu.async_copy(rows_v.at[b],
                             out_nbr.at[pl.ds((base + l) * CHN, CHN)],
                             sem_o[b])

        def drain_out(b):
            pltpu.make_async_copy(rows_v.at[b], out_nbr.at[pl.ds(0, CHN)],
                                  sem_o[b]).wait()

        for b in range(NBUF):
            issue_gather(b, b)

        def step(i, carry):
            first = NBUF * i
            for b in range(NBUF):
                l = first + b
                wait_gather(b)
                put(l, b)
                nxt = l + NBUF

                @pl.when(nxt < nch)
                def _next():
                    drain_out(b)
                    issue_gather(nxt, b)

            return carry

        lax.fori_loop(0, nch // NBUF, step, 0)
        for b in range(NBUF):
            drain_out(b)

        for t in range(cch):
            pltpu.async_copy(table_f32.at[cidx_v.at[t]], cen_v, sem_c).wait()
            pltpu.sync_copy(cen_v,
                            out_cen.at[pl.ds((w * cch + t) * CHN, CHN)])

    return _sc_gather


def _agg_body(cen_ref, g1_ref, g2_ref, lab_ref, ws_ref, wn_ref, wn2_ref,
              w1_ref, b1_ref, w2_ref, b2_ref, raw_ref, csum_ref):
    i = pl.program_id(0)
    c = ws_ref[...] * cen_ref[...]

    def unpack(x):
        lo = lax.bitcast_convert_type(x << 16, jnp.float32)
        hi = lax.bitcast_convert_type(x & jnp.int32(-65536), jnp.float32)
        return jnp.concatenate([lo, hi], axis=-1)

    g1 = wn_ref[...][:, None, :] * unpack(g1_ref[...]).reshape(BB1, K1, D)
    g2 = wn2_ref[...][:, None, :] * unpack(g2_ref[...]).reshape(BB1, K2, D)
    na = jnp.maximum(jnp.sqrt(jnp.sum(c * c, axis=-1)), EPS)
    n1 = jnp.maximum(jnp.sqrt(jnp.sum(g1 * g1, axis=-1)), EPS)
    n2 = jnp.maximum(jnp.sqrt(jnp.sum(g2 * g2, axis=-1)), EPS)
    d1 = jnp.sum(c[:, None, :] * g1, axis=-1)
    d2 = jnp.sum(c[:, None, :] * g2, axis=-1)
    s1 = d1 / (na[:, None] * n1)
    s2 = d2 / (na[:, None] * n2)
    m = jnp.maximum(jnp.max(s1, axis=-1), jnp.max(s2, axis=-1))
    e1 = jnp.exp(s1 - m[:, None])
    e2 = jnp.exp(s2 - m[:, None])
    z = jnp.sum(e1, axis=-1) + jnp.sum(e2, axis=-1)
    p1 = e1 / z[:, None]
    p2 = e2 / z[:, None]
    wsum = (jnp.sum(p1[:, :, None] * g1, axis=1)
            + jnp.sum(p2[:, :, None] * g2, axis=1))
    x = wsum + c
    h = jnp.maximum(
        jnp.dot(x, w1_ref[...], preferred_element_type=jnp.float32)
        + b1_ref[...], 0.0)
    pr = jnp.dot(h, w2_ref[...], preferred_element_type=jnp.float32) + b2_ref[...]
    raw = pr + c
    raw_ref[...] = raw
    lab = lab_ref[0]
    cls = lax.broadcasted_iota(jnp.int32, (8, BB1), 0)
    pmat = (cls == lab).astype(jnp.float32)
    part = jnp.dot(pmat, raw, preferred_element_type=jnp.float32)

    @pl.when(i == 0)
    def _init():
        csum_ref[...] = jnp.zeros_like(csum_ref)

    csum_ref[...] += part


def _agg_call(gnbr, cen, cen_blk, hb, lab3, w_self, w_nbr, w_nbr2,
              W1, b1, W2, b2):
    full2 = lambda shape: pl.BlockSpec(shape, lambda i: (0, 0))
    n1_blk = (hb * K2) // (BB1 * K1)
    return pl.pallas_call(
        _agg_body,
        grid=(hb // BB1,),
        in_specs=[
            pl.BlockSpec((BB1, D), lambda i: (cen_blk + i, 0)),
            pl.BlockSpec((BB1 * K1, DW), lambda i: (n1_blk + i, 0)),
            pl.BlockSpec((BB1 * K2, DW), lambda i: (i, 0)),
            pl.BlockSpec((1, 1, BB1), lambda i: (i, 0, 0)),
            full2((1, D)), full2((1, D)), full2((1, D)),
            full2((D, BOT)), full2((1, BOT)), full2((BOT, D)), full2((1, D)),
        ],
        out_specs=[
            pl.BlockSpec((BB1, D), lambda i: (i, 0)),
            pl.BlockSpec((8, D), lambda i: (0, 0)),
        ],
        out_shape=[
            jax.ShapeDtypeStruct((hb, D), jnp.float32),
            jax.ShapeDtypeStruct((8, D), jnp.float32),
        ],
    )(cen, gnbr, gnbr, lab3, w_self, w_nbr, w_nbr2, W1, b1, W2, b2)


def _cos_body(raw_ref, csa_ref, csb_ref, o_ref):
    ave = (csa_ref[...] + csb_ref[...]) * (1.0 / PER_CLASS)
    r = raw_ref[...]
    dots = lax.dot_general(r, ave, (((1,), (1,)), ((), ())),
                           preferred_element_type=jnp.float32)
    na = jnp.maximum(jnp.sqrt(jnp.sum(r * r, axis=-1)), EPS)
    nb = jnp.maximum(jnp.sqrt(jnp.sum(ave * ave, axis=-1)), EPS)
    sim = dots / (na[:, None] * nb[None, :])
    col = lax.broadcasted_iota(jnp.int32, (BB2, 8), 1)
    sim = jnp.where(col < NB, sim, -1e30)
    m = jnp.max(sim, axis=-1)
    e = jnp.exp(sim - m[:, None])
    o_ref[...] = e / jnp.sum(e, axis=-1)[:, None]


def _cos_call(raw, csum_a, csum_b, hb):
    return pl.pallas_call(
        _cos_body,
        grid=(hb // BB2,),
        in_specs=[
            pl.BlockSpec((BB2, D), lambda i: (i, 0)),
            pl.BlockSpec((8, D), lambda i: (0, 0)),
            pl.BlockSpec((8, D), lambda i: (0, 0)),
        ],
        out_specs=pl.BlockSpec((BB2, 8), lambda i: (i, 0)),
        out_shape=jax.ShapeDtypeStruct((hb, 8), jnp.float32),
    )(raw, csum_a, csum_b)


def kernel(embeds, idx, neighbors, neighbors_2hop, labels, w_self, w_nbr,
           w_nbr2, W1, b1, W2, b2):
    pad = BP - B
    idxp = jnp.concatenate([idx, jnp.zeros((pad,), jnp.int32)])
    nbrp = jnp.concatenate([neighbors, jnp.zeros((pad, K1), jnp.int32)])
    nbr2p = jnp.concatenate([neighbors_2hop, jnp.zeros((pad, K2), jnp.int32)])
    labp = jnp.concatenate([labels, jnp.full((pad,), NB, jnp.int32)])

    emb_bf = embeds.astype(jnp.bfloat16)
    emb_i32 = lax.bitcast_convert_type(
        jnp.stack([emb_bf[:, :DW], emb_bf[:, DW:]], axis=-1), jnp.int32)

    b1r = b1.reshape(1, BOT)
    b2r = b2.reshape(1, D)

    def nbr_idx(lo, hb):
        return jnp.concatenate(
            [nbr2p[lo:lo + hb].reshape(hb * K2),
             nbrp[lo:lo + hb].reshape(hb * K1)]).reshape(-1, CHN)

    idxc2d = idxp.reshape(BP // CHN, CHN)
    gnbr_a, cen = _sc_gather_build(HBA, BP)(
        emb_i32, embeds, nbr_idx(0, HBA), idxc2d)
    lab3_a = labp[:HBA].reshape(HBA // BB1, 1, BB1)
    raw_a, cs_a = _agg_call(gnbr_a, cen, 0, HBA, lab3_a, w_self, w_nbr,
                            w_nbr2, W1, b1r, W2, b2r)
    if HBB:
        res_b = _sc_gather_build(HBB, 0)(emb_i32, embeds, nbr_idx(HBA, HBB))
        gnbr_b = res_b[0] if isinstance(res_b, (list, tuple)) else res_b
        lab3_b = labp[HBA:].reshape(HBB // BB1, 1, BB1)
        raw_b, cs_b = _agg_call(gnbr_b, cen, HBA // BB1, HBB, lab3_b, w_self,
                                w_nbr, w_nbr2, W1, b1r, W2, b2r)
        out_a = _cos_call(raw_a, cs_a, cs_b, HBA)
        out_b = _cos_call(raw_b, cs_a, cs_b, HBB)
        out = jnp.concatenate([out_a, out_b], axis=0)
    else:
        out = _cos_call(raw_a, cs_a, jnp.zeros((8, D), jnp.float32), HBA)
    return out[:B, :NB]
